# Initial kernel scaffold; baseline (speedup 1.0000x reference)
import jax
import jax.numpy as jnp
from jax.experimental import pallas as pl


def kernel(token_ids, weight):
    raise NotImplementedError("write your pallas kernel here")


# speedup vs baseline: 1.8497x; 1.8497x over previous
import functools

import jax
import jax.numpy as jnp
from jax import lax
from jax.experimental import pallas as pl
from jax.experimental.pallas import tpu as pltpu
from jax.experimental.pallas import tpu_sc as plsc


@functools.cache
def _build_gather(B: int, V: int, D: int):
    info = plsc.get_sparse_core_info()
    NC, NS = info.num_cores, info.num_subcores
    NW = NC * NS
    assert B % NW == 0
    b_per_w = B // NW
    CHUNK = 800
    assert b_per_w % CHUNK == 0
    n_chunks = b_per_w // CHUNK

    mesh = plsc.VectorSubcoreMesh(core_axis_name="c", subcore_axis_name="s")

    @functools.partial(
        pl.kernel,
        mesh=mesh,
        out_type=jax.ShapeDtypeStruct((B, D), jnp.float32),
        scratch_types=[
            pltpu.VMEM((b_per_w,), jnp.int32),
            pltpu.VMEM((CHUNK, D), jnp.float32),
            pltpu.SemaphoreType.DMA,
        ],
        compiler_params=pltpu.CompilerParams(use_tc_tiling_on_sc=False),
    )
    def gather_kernel(idx_hbm, table_hbm, out_hbm, idx_v, rows_v, sem):
        wid = lax.axis_index("s") * NC + lax.axis_index("c")
        base = wid * b_per_w
        pltpu.sync_copy(idx_hbm.at[pl.ds(base, b_per_w)], idx_v)

        def chunk_body(g, carry):
            off = pl.multiple_of(g * CHUNK, 8)
            pltpu.async_copy(
                table_hbm.at[idx_v.at[pl.ds(off, CHUNK)]], rows_v, sem
            ).wait()
            pltpu.sync_copy(rows_v, out_hbm.at[pl.ds(base + off, CHUNK)])
            return carry

        lax.fori_loop(0, n_chunks, chunk_body, 0)

    return gather_kernel


def kernel(token_ids, weight):
    V, D = weight.shape
    flat = token_ids.reshape(-1).astype(jnp.int32)
    out = _build_gather(flat.shape[0], V, D)(flat, weight)
    return out.reshape(*token_ids.shape, D)


# speedup vs baseline: 1.8749x; 1.0137x over previous
import functools

import jax
import jax.numpy as jnp
from jax import lax
from jax.experimental import pallas as pl
from jax.experimental.pallas import tpu as pltpu
from jax.experimental.pallas import tpu_sc as plsc


@functools.cache
def _build_gather(B: int, V: int, D: int):
    info = plsc.get_sparse_core_info()
    NC, NS = info.num_cores, info.num_subcores
    NW = NC * NS
    assert B % NW == 0
    b_per_w = B // NW
    CHUNK = 800
    assert b_per_w % CHUNK == 0
    n_chunks = b_per_w // CHUNK

    mesh = plsc.VectorSubcoreMesh(core_axis_name="c", subcore_axis_name="s")

    assert n_chunks >= 4 and n_chunks % 2 == 0

    @functools.partial(
        pl.kernel,
        mesh=mesh,
        out_type=jax.ShapeDtypeStruct((B, D), jnp.float32),
        scratch_types=[
            pltpu.VMEM((b_per_w,), jnp.int32),
            pltpu.VMEM((2, CHUNK, D), jnp.float32),
            pltpu.SemaphoreType.DMA,
            pltpu.SemaphoreType.DMA,
        ],
        compiler_params=pltpu.CompilerParams(use_tc_tiling_on_sc=False),
    )
    def gather_kernel(idx_hbm, table_hbm, out_hbm, idx_v, rows_v, gsem, wsem):
        wid = lax.axis_index("s") * NC + lax.axis_index("c")
        base = wid * b_per_w
        pltpu.sync_copy(idx_hbm.at[pl.ds(base, b_per_w)], idx_v)

        def gather_chunk(g, b):
            off = pl.multiple_of(g * CHUNK, 8)
            return pltpu.make_async_copy(
                table_hbm.at[idx_v.at[pl.ds(off, CHUNK)]], rows_v.at[b], gsem
            )

        def write_chunk(g, b):
            off = pl.multiple_of(g * CHUNK, 8)
            return pltpu.make_async_copy(
                rows_v.at[b], out_hbm.at[pl.ds(base + off, CHUNK)], wsem
            )

        gather_chunk(0, 0).start()
        gather_chunk(1, 1).start()
        gather_chunk(0, 0).wait()
        write_chunk(0, 0).start()

        def body(i, carry):
            for b, g in ((1, 2 * i + 1), (0, 2 * i + 2)):
                write_chunk(g - 1, 1 - b).wait()
                gather_chunk(g + 1, 1 - b).start()
                gather_chunk(g, b).wait()
                write_chunk(g, b).start()
            return carry

        lax.fori_loop(0, (n_chunks - 2) // 2, body, 0)

        g_last = n_chunks - 1
        write_chunk(g_last - 1, 0).wait()
        gather_chunk(g_last, 1).wait()
        write_chunk(g_last, 1).start()
        write_chunk(g_last, 1).wait()

    return gather_kernel


def kernel(token_ids, weight):
    V, D = weight.shape
    flat = token_ids.reshape(-1).astype(jnp.int32)
    out = _build_gather(flat.shape[0], V, D)(flat, weight)
    return out.reshape(*token_ids.shape, D)


# speedup vs baseline: 1.9750x; 1.0534x over previous
import functools

import jax
import jax.numpy as jnp
from jax import lax
from jax.experimental import pallas as pl
from jax.experimental.pallas import tpu as pltpu
from jax.experimental.pallas import tpu_sc as plsc


@functools.cache
def _build_gather(B: int, V: int, D: int):
    info = plsc.get_sparse_core_info()
    NC, NS = info.num_cores, info.num_subcores
    NW = NC * NS
    assert B % NW == 0
    b_per_w = B // NW
    CHUNK = 800
    assert b_per_w % CHUNK == 0
    n_chunks = b_per_w // CHUNK

    mesh = plsc.VectorSubcoreMesh(core_axis_name="c", subcore_axis_name="s")

    assert n_chunks >= 4 and n_chunks % 2 == 0

    @functools.partial(
        pl.kernel,
        mesh=mesh,
        out_type=jax.ShapeDtypeStruct((B, D), jnp.float32),
        scratch_types=[
            pltpu.VMEM((b_per_w,), jnp.int32),
            pltpu.VMEM((2, CHUNK, D), jnp.float32),
            pltpu.SemaphoreType.DMA,
            pltpu.SemaphoreType.DMA,
        ],
        compiler_params=pltpu.CompilerParams(use_tc_tiling_on_sc=False),
    )
    def gather_kernel(idx_hbm, table_hbm, out_hbm, idx_v, rows_v, gsem, wsem):
        wid = lax.axis_index("s") * NC + lax.axis_index("c")
        base = wid * b_per_w
        pltpu.sync_copy(idx_hbm.at[pl.ds(base, b_per_w)], idx_v)

        def gather_chunk(g, b):
            off = pl.multiple_of(g * CHUNK, 8)
            return pltpu.make_async_copy(
                table_hbm.at[idx_v.at[pl.ds(off, CHUNK)]], rows_v.at[b], gsem
            )

        def write_chunk(g, b):
            off = pl.multiple_of(g * CHUNK, 8)
            return pltpu.make_async_copy(
                rows_v.at[b], out_hbm.at[pl.ds(base + off, CHUNK)], wsem
            )

        gather_chunk(0, 0).start()
        gather_chunk(1, 1).start()
        gather_chunk(0, 0).wait()
        write_chunk(0, 0).start()

        def body(i, carry):
            for b, g in ((1, 2 * i + 1), (0, 2 * i + 2)):
                write_chunk(g - 1, 1 - b).wait()
                gather_chunk(g + 1, 1 - b).start()
                gather_chunk(g, b).wait()
                write_chunk(g, b).start()
            return carry

        lax.fori_loop(0, (n_chunks - 2) // 2, body, 0)

        g_last = n_chunks - 1
        write_chunk(g_last - 1, 0).wait()
        gather_chunk(g_last, 1).wait()
        write_chunk(g_last, 1).start()
        write_chunk(g_last, 1).wait()

    return gather_kernel


def kernel(token_ids, weight):
    V, D = weight.shape
    wpad = jnp.pad(weight, ((0, 0), (0, D))).reshape(2 * V, D)
    flat = token_ids.reshape(-1).astype(jnp.int32) * 2
    out = _build_gather(flat.shape[0], 2 * V, D)(flat, wpad)
    return out.reshape(*token_ids.shape, D)
